# baseline (speedup 1.0000x reference)
import functools

import jax
import jax.numpy as jnp
from jax import lax
from jax.experimental import pallas as pl
from jax.experimental.pallas import tpu as pltpu
from jax.experimental.pallas import tpu_sc as plsc

H = 1024
ROW0 = 248
NROWS = 264
COL0 = 128
NCOLS = 384
TROWS = NROWS + 1
NW = 32
ROWS_W = H // NW
CROWS = 2
NCHUNK = ROWS_W // CROWS
VECS = CROWS * H // 16

_mesh = plsc.VectorSubcoreMesh(core_axis_name="c", subcore_axis_name="s")


@functools.partial(
    pl.kernel,
    mesh=_mesh,
    out_type=jax.ShapeDtypeStruct((H, H), jnp.float32),
    scratch_types=[
        pltpu.VMEM((TROWS, NCOLS), jnp.float32),
        pltpu.VMEM((CROWS, 16, 128), jnp.float32),
        pltpu.VMEM((CROWS, 16, 128), jnp.float32),
        pltpu.VMEM((CROWS, H), jnp.float32),
        pltpu.VMEM((CROWS, H), jnp.float32),
        pltpu.SemaphoreType.DMA,
        pltpu.SemaphoreType.DMA,
        pltpu.SemaphoreType.DMA,
        pltpu.SemaphoreType.DMA,
        pltpu.SemaphoreType.DMA,
    ],
    compiler_params=pltpu.CompilerParams(needs_layout_passes=False),
)
def _sample(x_hbm, grid_hbm, out_hbm, tab_v, cin_a, cin_b, cout_a, cout_b,
            tab_sem, in_sem_a, in_sem_b, out_sem_a, out_sem_b):
    wid = lax.axis_index("s") * 2 + lax.axis_index("c")
    base_row = wid * ROWS_W
    cins = (cin_a, cin_b)
    couts = (cout_a, cout_b)
    in_sems = (in_sem_a, in_sem_b)
    out_sems = (out_sem_a, out_sem_b)
    zerosv = jnp.zeros((16,), jnp.float32)

    def start_in(c):
        return pltpu.async_copy(
            x_hbm.at[pl.ds(base_row + c * CROWS, CROWS)], cins[c % 2],
            in_sems[c % 2])

    in_pending = [start_in(0), start_in(1)]
    tab_pending = pltpu.async_copy(
        grid_hbm.at[pl.ds(ROW0, NROWS), pl.ds(COL0, NCOLS)],
        tab_v.at[pl.ds(0, NROWS)], tab_sem)
    for k in range(NCOLS // 16):
        tab_v[NROWS, pl.ds(k * 16, 16)] = zerosv
    tab_pending.wait()

    out_pending = [None, None]
    for c in range(NCHUNK):
        cin_v = cins[c % 2]
        cout_v = couts[c % 2]
        if out_pending[c % 2] is not None:
            out_pending[c % 2].wait()
        in_pending[c % 2].wait()

        @plsc.parallel_loop(0, VECS, 1, unroll=8)
        def vec_body(j):
            r = j >> 6
            u = j & 63
            kb = (u >> 3) * 2
            wl0 = (u & 7) * 16
            gx = cin_v[r, kb, pl.ds(wl0, 16)]
            gy = cin_v[r, kb + 1, pl.ds(wl0, 16)]
            ix = gx * 256.0 + (255.5 - COL0)
            iy = gy * 256.0 + (255.5 - ROW0)
            cx = ix.astype(jnp.int32)
            dy = iy.astype(jnp.int32)
            fx = ix - cx.astype(jnp.float32)
            fy = iy - dy.astype(jnp.float32)
            inx = cx < (NCOLS - 1)
            cx1 = jnp.minimum(cx + 1, NCOLS - 1)
            dy1 = dy + 1
            v00 = plsc.load_gather(tab_v, [dy, cx])
            v01 = plsc.load_gather(tab_v, [dy, cx1])
            v10 = plsc.load_gather(tab_v, [dy1, cx])
            v11 = plsc.load_gather(tab_v, [dy1, cx1])
            v01 = jnp.where(inx, v01, 0.0)
            v11 = jnp.where(inx, v11, 0.0)
            m0 = v00 + fx * (v01 - v00)
            m1 = v10 + fx * (v11 - v10)
            res = m0 + fy * (m1 - m0)
            cout_v[r, pl.ds(u * 16, 16)] = res

        out_pending[c % 2] = pltpu.async_copy(
            cout_v, out_hbm.at[pl.ds(base_row + c * CROWS, CROWS)],
            out_sems[c % 2])
        if c + 2 < NCHUNK:
            in_pending[c % 2] = start_in(c + 2)

    out_pending[0].wait()
    out_pending[1].wait()


def kernel(x, grid):
    xv = x.reshape(H, 8, 128, 2).transpose(0, 1, 3, 2).reshape(H, 16, 128)
    g2 = grid.reshape(512, 512)
    out = _sample(xv, g2)
    return out.reshape(1, 1, H, H)


# speedup vs baseline: 1.2318x; 1.2318x over previous
import functools

import jax
import jax.numpy as jnp
from jax import lax
from jax.experimental import pallas as pl
from jax.experimental.pallas import tpu as pltpu
from jax.experimental.pallas import tpu_sc as plsc

H = 1024
ROW0 = 248
NROWS = 264
COL0 = 128
NCOLS = 384
TSTRIDE = 392
TSIZE = 103936
NBLK = 33
BLKW = 8 * TSTRIDE
NW = 32
ROWS_W = H // NW
CROWS = 2
NCHUNK = ROWS_W // CROWS
VECS = CROWS * H // 16

_mesh = plsc.VectorSubcoreMesh(core_axis_name="c", subcore_axis_name="s")


@functools.partial(
    pl.kernel,
    mesh=_mesh,
    out_type=(
        jax.ShapeDtypeStruct((H, H), jnp.float32),
        jax.ShapeDtypeStruct((2 * TSIZE,), jnp.float32),
    ),
    scratch_types=[
        pltpu.VMEM((TSIZE,), jnp.float32),
        pltpu.VMEM((3072,), jnp.float32),
        pltpu.VMEM((3152,), jnp.float32),
        pltpu.VMEM((CROWS, 16, 128), jnp.float32),
        pltpu.VMEM((CROWS, 16, 128), jnp.float32),
        pltpu.VMEM((CROWS, H), jnp.float32),
        pltpu.VMEM((CROWS, H), jnp.float32),
        pltpu.SemaphoreType.DMA,
        pltpu.SemaphoreType.DMA,
        pltpu.SemaphoreType.DMA,
        pltpu.SemaphoreType.DMA,
        pltpu.SemaphoreType.DMA,
    ],
    compiler_params=pltpu.CompilerParams(needs_layout_passes=False),
)
def _sample(x_hbm, gflat_hbm, out_hbm, tmp_hbm, tab_v, stage_v, blk_v,
            cin_a, cin_b, cout_a, cout_b,
            tab_sem, in_sem_a, in_sem_b, out_sem_a, out_sem_b):
    core = lax.axis_index("c")
    sid = lax.axis_index("s")
    wid = sid * 2 + core
    base_row = wid * ROWS_W
    cins = (cin_a, cin_b)
    couts = (cout_a, cout_b)
    in_sems = (in_sem_a, in_sem_b)
    out_sems = (out_sem_a, out_sem_b)
    zerosv = jnp.zeros((16,), jnp.float32)
    tmp_base = core * TSIZE

    def start_in(c):
        return pltpu.async_copy(
            x_hbm.at[pl.ds(base_row + c * CROWS, CROWS)], cins[c % 2],
            in_sems[c % 2])

    in_pending = [start_in(0), start_in(1)]

    def build_block(b):
        src_off = pl.multiple_of((125 + 4 * b) * 1024, 1024)
        pltpu.sync_copy(gflat_hbm.at[pl.ds(src_off, 3072)], stage_v)
        for ri in range(8):
            blk_v[pl.ds(ri * TSTRIDE + NCOLS, 16)] = zerosv
        for ri in range(8):
            for tc in range(3):
                for c in range(8):
                    blk_v[pl.ds(ri * TSTRIDE + tc * 128 + c * 16, 16)] = (
                        stage_v[pl.ds(tc * 1024 + ri * 128 + c * 16, 16)])
        dst_off = pl.multiple_of(tmp_base + b * BLKW, 8)
        return pltpu.async_copy(
            blk_v.at[pl.ds(0, BLKW)], tmp_hbm.at[pl.ds(dst_off, BLKW)],
            tab_sem)

    h0 = build_block(sid)
    h0.wait()
    h1 = build_block(sid + 16)
    h1.wait()

    @pl.when(sid == 0)
    def _():
        build_block(jnp.int32(32)).wait()

    plsc.subcore_barrier()

    big = pltpu.async_copy(tmp_hbm.at[pl.ds(tmp_base, TSIZE)], tab_v, tab_sem)
    big.wait()
    for k in range(25):
        tab_v[pl.ds(NROWS * TSTRIDE + k * 16, 16)] = zerosv

    out_pending = [None, None]
    for c in range(NCHUNK):
        cin_v = cins[c % 2]
        cout_v = couts[c % 2]
        if out_pending[c % 2] is not None:
            out_pending[c % 2].wait()
        in_pending[c % 2].wait()

        @plsc.parallel_loop(0, VECS, 1, unroll=8)
        def vec_body(j):
            r = j >> 6
            u = j & 63
            kb = (u >> 3) * 2
            wl0 = (u & 7) * 16
            gx = cin_v[r, kb, pl.ds(wl0, 16)]
            gy = cin_v[r, kb + 1, pl.ds(wl0, 16)]
            ix = gx * 256.0 + (255.5 - COL0)
            iy = gy * 256.0 + (255.5 - ROW0)
            cx = ix.astype(jnp.int32)
            dy = iy.astype(jnp.int32)
            fx = ix - cx.astype(jnp.float32)
            fy = iy - dy.astype(jnp.float32)
            i00 = dy * TSTRIDE + cx
            v00 = plsc.load_gather(tab_v, [i00])
            v01 = plsc.load_gather(tab_v, [i00 + 1])
            v10 = plsc.load_gather(tab_v, [i00 + TSTRIDE])
            v11 = plsc.load_gather(tab_v, [i00 + (TSTRIDE + 1)])
            m0 = v00 + fx * (v01 - v00)
            m1 = v10 + fx * (v11 - v10)
            res = m0 + fy * (m1 - m0)
            cout_v[r, pl.ds(u * 16, 16)] = res

        out_pending[c % 2] = pltpu.async_copy(
            cout_v, out_hbm.at[pl.ds(base_row + c * CROWS, CROWS)],
            out_sems[c % 2])
        if c + 2 < NCHUNK:
            in_pending[c % 2] = start_in(c + 2)

    out_pending[0].wait()
    out_pending[1].wait()


def kernel(x, grid):
    xv = x.reshape(H, 8, 128, 2).transpose(0, 1, 3, 2).reshape(H, 16, 128)
    gflat = grid.reshape(64, 8, 4, 128).transpose(0, 2, 1, 3).reshape(-1)
    out, _ = _sample(xv, gflat)
    return out.reshape(1, 1, H, H)
